# Initial kernel scaffold; baseline (speedup 1.0000x reference)
import jax
import jax.numpy as jnp
from jax.experimental import pallas as pl


def kernel(x, edge_index, edge_attr, W0, b0, We0, W1, b1, We1, W2, b2, We2):
    raise NotImplementedError("write your pallas kernel here")


# speedup vs baseline: 5.1698x; 5.1698x over previous
import functools

import jax
import jax.numpy as jnp
from jax import lax
from jax.experimental import pallas as pl
from jax.experimental.pallas import tpu as pltpu
from jax.experimental.pallas import tpu_sc as plsc

_N = 10000
_E = 320000
_D = 128
_DE = 4
_NC, _NS = 2, 16
_NW = _NC * _NS
_EPW = _E // _NW
_B = 80
_NB = _EPW // _B
_CH = 25
_NCH = _NB // _CH
_NP = 10240
_RPT = _NP // _NS
_ZR = 8


def _sc_mesh():
    return plsc.VectorSubcoreMesh(core_axis_name="c", subcore_axis_name="s",
                                  num_cores=_NC, num_subcores=_NS)


_DP = 16


@functools.cache
def _make_sc():
    out_type = [jax.ShapeDtypeStruct((_NC, _NP, _D), jnp.float32)]
    scratch = [
        pltpu.VMEM_SHARED((_NP, _D), jnp.float32),
        pltpu.VMEM((_CH, _B), jnp.int32),
        pltpu.VMEM((_CH, _B), jnp.int32),
        pltpu.VMEM((_B, _D), jnp.float32),
        pltpu.VMEM((_ZR, _D), jnp.float32),
        pltpu.SemaphoreType.DMA,
    ]

    def body(g, src3, dst3, accout,
             acc_sh, src_v, dst_v, rows_v, zbuf, sem):
        c = lax.axis_index("c")
        s = lax.axis_index("s")
        wid = s * _NC + c
        rowbase = s * _RPT
        z = jnp.zeros((16,), jnp.float32)

        @pl.loop(0, _ZR)
        def _zero(i):
            for j in range(_D // 16):
                zbuf[i, pl.ds(j * 16, 16)] = z

        for k in range(_RPT // _ZR):
            pltpu.sync_copy(zbuf, acc_sh.at[pl.ds(rowbase + _ZR * k, _ZR)])

        plsc.subcore_barrier()

        @pl.loop(0, _NCH)
        def _chunks(k):
            pltpu.sync_copy(src3.at[wid, k], src_v)
            pltpu.sync_copy(dst3.at[wid, k], dst_v)

            @pl.loop(0, _CH)
            def _edges(j):
                pltpu.async_copy(g.at[src_v.at[j]], rows_v, sem).wait()
                pltpu.sync_copy(rows_v, acc_sh.at[dst_v.at[j]], add=True)

        plsc.subcore_barrier()
        pltpu.sync_copy(acc_sh.at[pl.ds(rowbase, _RPT)],
                        accout.at[c, pl.ds(rowbase, _RPT)])

    return pl.kernel(body, out_type=out_type, mesh=_sc_mesh(),
                     scratch_types=scratch)


@functools.cache
def _make_ea():
    out_type = [jax.ShapeDtypeStruct((_NC, _NP, _D), jnp.float32)]
    scratch = [
        pltpu.VMEM_SHARED((_NP, _D), jnp.float32),
        pltpu.VMEM((_CH, _B), jnp.int32),
        pltpu.VMEM((_B, _D), jnp.float32),
        pltpu.VMEM((_ZR, _D), jnp.float32),
    ]

    def body(ea3, dst3, eaout, ea_sh, dst_v, eab_v, zbuf):
        c = lax.axis_index("c")
        s = lax.axis_index("s")
        wid = s * _NC + c
        rowbase = s * _RPT
        z = jnp.zeros((16,), jnp.float32)

        @pl.loop(0, _ZR)
        def _zero(i):
            for j in range(_D // 16):
                zbuf[i, pl.ds(j * 16, 16)] = z

        for k in range(_RPT // _ZR):
            pltpu.sync_copy(zbuf, ea_sh.at[pl.ds(rowbase + _ZR * k, _ZR)])
        plsc.subcore_barrier()

        @pl.loop(0, _NCH)
        def _chunks(k):
            pltpu.sync_copy(dst3.at[wid, k], dst_v)

            @pl.loop(0, _CH)
            def _edges(j):
                pltpu.sync_copy(ea3.at[wid, k * _CH + j], eab_v)
                pltpu.sync_copy(eab_v, ea_sh.at[dst_v.at[j]], add=True)

        plsc.subcore_barrier()
        pltpu.sync_copy(ea_sh.at[pl.ds(rowbase, _RPT)],
                        eaout.at[c, pl.ds(rowbase, _RPT)])

    return pl.kernel(body, out_type=out_type, mesh=_sc_mesh(),
                     scratch_types=scratch)


_RB = 1000


@functools.cache
def _make_tc(scale: float, pooled: bool):
    if pooled:
        out_shape = jax.ShapeDtypeStruct((1, _D), jnp.float32)
        out_spec = pl.BlockSpec((1, _D), lambda i: (0, 0))
    else:
        out_shape = jax.ShapeDtypeStruct((_N, _D), jnp.float32)
        out_spec = pl.BlockSpec((_RB, _D), lambda i: (i, 0))

    def body(acc_ref, ea_ref, We_ref, W_ref, b_ref, o_ref):
        a = acc_ref[0] + acc_ref[1]
        e = (ea_ref[0] + ea_ref[1])[:, :_DE]
        Wm = W_ref[...]
        WeW = jnp.dot(We_ref[...], Wm, preferred_element_type=jnp.float32)
        y = scale * jnp.dot(a, Wm, preferred_element_type=jnp.float32)
        y = y + jnp.dot(e, WeW, preferred_element_type=jnp.float32) + b_ref[...]
        h = jnp.where(y >= 0, y, 0.2 * y)
        if pooled:
            ps = jnp.sum(h, axis=0, keepdims=True)

            @pl.when(pl.program_id(0) == 0)
            def _first():
                o_ref[...] = ps

            @pl.when(pl.program_id(0) != 0)
            def _rest():
                o_ref[...] += ps
        else:
            o_ref[...] = h

    return pl.pallas_call(
        body,
        grid=(_N // _RB,),
        in_specs=[
            pl.BlockSpec((_NC, _RB, _D), lambda i: (0, i, 0)),
            pl.BlockSpec((_NC, _RB, _D), lambda i: (0, i, 0)),
            pl.BlockSpec((_DE, _D), lambda i: (0, 0)),
            pl.BlockSpec((_D, _D), lambda i: (0, 0)),
            pl.BlockSpec((1, _D), lambda i: (0, 0)),
        ],
        out_specs=out_spec,
        out_shape=out_shape,
    )


def kernel(x, edge_index, edge_attr, W0, b0, We0, W1, b1, We1, W2, b2, We2):
    src3 = edge_index[0].reshape(_NW, _NCH, _CH, _B)
    dst3 = edge_index[1].reshape(_NW, _NCH, _CH, _B)
    ea128 = jnp.pad(edge_attr, ((0, 0), (0, _D - _DE)))
    ea3 = ea128.reshape(_NW, _NB, _B, _D)

    (eap,) = _make_ea()(ea3, dst3)
    (acc0,) = _make_sc()(x, src3, dst3)
    h0 = _make_tc(1.0, False)(acc0, eap, We0, W0, b0.reshape(1, _D))
    (acc1,) = _make_sc()(h0, src3, dst3)
    h1 = _make_tc(2.0, False)(acc1, eap, We1, W1, b1.reshape(1, _D))
    (acc2,) = _make_sc()(h1, src3, dst3)
    return _make_tc(2.0, True)(acc2, eap, We2, W2, b2.reshape(1, _D))


# speedup vs baseline: 5.7332x; 1.1090x over previous
import functools

import jax
import jax.numpy as jnp
from jax import lax
from jax.experimental import pallas as pl
from jax.experimental.pallas import tpu as pltpu
from jax.experimental.pallas import tpu_sc as plsc

_N = 10000
_E = 320000
_D = 128
_DE = 4
_NC, _NS = 2, 16
_NW = _NC * _NS
_EPW = _E // _NW
_B = 80
_NB = _EPW // _B
_CH = 25
_NCH = _NB // _CH
_B2 = 40
_NB2 = _EPW // _B2
_CH2 = 10
_NCH2 = _NB2 // _CH2
_NP = 10240
_RPT = _NP // _NS
_ZR = 8


def _sc_mesh():
    return plsc.VectorSubcoreMesh(core_axis_name="c", subcore_axis_name="s",
                                  num_cores=_NC, num_subcores=_NS)


_DP = 16


@functools.cache
def _make_sc():
    out_type = [jax.ShapeDtypeStruct((_NC, _NP, _D), jnp.float32)]
    scratch = [
        pltpu.VMEM_SHARED((_NP, _D), jnp.float32),
        pltpu.VMEM((_NB2, _B2), jnp.int32),
        pltpu.VMEM((_CH2, _B2), jnp.int32),
        pltpu.VMEM((_B2, _D), jnp.float32),
        pltpu.VMEM((_B2, _D), jnp.float32),
        pltpu.VMEM((_ZR, _D), jnp.float32),
        pltpu.SemaphoreType.DMA,
        pltpu.SemaphoreType.DMA,
    ]

    def body(g, src2, dst3, accout,
             acc_sh, src_v, dst_v, rows0, rows1, zbuf, sem0, sem1):
        c = lax.axis_index("c")
        s = lax.axis_index("s")
        wid = s * _NC + c
        rowbase = s * _RPT
        z = jnp.zeros((16,), jnp.float32)

        @pl.loop(0, _ZR)
        def _zero(i):
            for j in range(_D // 16):
                zbuf[i, pl.ds(j * 16, 16)] = z

        for k in range(_RPT // _ZR):
            pltpu.sync_copy(zbuf, acc_sh.at[pl.ds(rowbase + _ZR * k, _ZR)])

        pltpu.sync_copy(src2.at[wid], src_v)
        plsc.subcore_barrier()

        pltpu.async_copy(g.at[src_v.at[0]], rows0, sem0)

        @pl.loop(0, _NCH2)
        def _chunks(k):
            pltpu.sync_copy(dst3.at[wid, k], dst_v)

            @pl.loop(0, _CH2 // 2)
            def _pairs(m):
                jj = k * _CH2 + 2 * m
                pltpu.async_copy(g.at[src_v.at[jj + 1]], rows1, sem1)
                pltpu.make_async_copy(g.at[src_v.at[0]], rows0, sem0).wait()
                pltpu.sync_copy(rows0, acc_sh.at[dst_v.at[2 * m]], add=True)

                @pl.when(jj + 2 < _NB2)
                def _():
                    pltpu.async_copy(g.at[src_v.at[jj + 2]], rows0, sem0)

                pltpu.make_async_copy(g.at[src_v.at[0]], rows1, sem1).wait()
                pltpu.sync_copy(rows1, acc_sh.at[dst_v.at[2 * m + 1]],
                                add=True)

        plsc.subcore_barrier()
        pltpu.sync_copy(acc_sh.at[pl.ds(rowbase, _RPT)],
                        accout.at[c, pl.ds(rowbase, _RPT)])

    return pl.kernel(body, out_type=out_type, mesh=_sc_mesh(),
                     scratch_types=scratch)


@functools.cache
def _make_ea():
    out_type = [jax.ShapeDtypeStruct((_NC, _NP, _D), jnp.float32)]
    scratch = [
        pltpu.VMEM_SHARED((_NP, _D), jnp.float32),
        pltpu.VMEM((_CH, _B), jnp.int32),
        pltpu.VMEM((_B, _D), jnp.float32),
        pltpu.VMEM((_ZR, _D), jnp.float32),
    ]

    def body(ea3, dst3, eaout, ea_sh, dst_v, eab_v, zbuf):
        c = lax.axis_index("c")
        s = lax.axis_index("s")
        wid = s * _NC + c
        rowbase = s * _RPT
        z = jnp.zeros((16,), jnp.float32)

        @pl.loop(0, _ZR)
        def _zero(i):
            for j in range(_D // 16):
                zbuf[i, pl.ds(j * 16, 16)] = z

        for k in range(_RPT // _ZR):
            pltpu.sync_copy(zbuf, ea_sh.at[pl.ds(rowbase + _ZR * k, _ZR)])
        plsc.subcore_barrier()

        @pl.loop(0, _NCH)
        def _chunks(k):
            pltpu.sync_copy(dst3.at[wid, k], dst_v)

            @pl.loop(0, _CH)
            def _edges(j):
                pltpu.sync_copy(ea3.at[wid, k * _CH + j], eab_v)
                pltpu.sync_copy(eab_v, ea_sh.at[dst_v.at[j]], add=True)

        plsc.subcore_barrier()
        pltpu.sync_copy(ea_sh.at[pl.ds(rowbase, _RPT)],
                        eaout.at[c, pl.ds(rowbase, _RPT)])

    return pl.kernel(body, out_type=out_type, mesh=_sc_mesh(),
                     scratch_types=scratch)


_RB = 1000


@functools.cache
def _make_tc(scale: float, pooled: bool):
    if pooled:
        out_shape = jax.ShapeDtypeStruct((1, _D), jnp.float32)
        out_spec = pl.BlockSpec((1, _D), lambda i: (0, 0))
    else:
        out_shape = jax.ShapeDtypeStruct((_N, _D), jnp.float32)
        out_spec = pl.BlockSpec((_RB, _D), lambda i: (i, 0))

    def body(acc_ref, ea_ref, We_ref, W_ref, b_ref, o_ref):
        a = acc_ref[0] + acc_ref[1]
        e = (ea_ref[0] + ea_ref[1])[:, :_DE]
        Wm = W_ref[...]
        WeW = jnp.dot(We_ref[...], Wm, preferred_element_type=jnp.float32)
        y = scale * jnp.dot(a, Wm, preferred_element_type=jnp.float32)
        y = y + jnp.dot(e, WeW, preferred_element_type=jnp.float32) + b_ref[...]
        h = jnp.where(y >= 0, y, 0.2 * y)
        if pooled:
            ps = jnp.sum(h, axis=0, keepdims=True)

            @pl.when(pl.program_id(0) == 0)
            def _first():
                o_ref[...] = ps

            @pl.when(pl.program_id(0) != 0)
            def _rest():
                o_ref[...] += ps
        else:
            o_ref[...] = h

    return pl.pallas_call(
        body,
        grid=(_N // _RB,),
        in_specs=[
            pl.BlockSpec((_NC, _RB, _D), lambda i: (0, i, 0)),
            pl.BlockSpec((_NC, _RB, _D), lambda i: (0, i, 0)),
            pl.BlockSpec((_DE, _D), lambda i: (0, 0)),
            pl.BlockSpec((_D, _D), lambda i: (0, 0)),
            pl.BlockSpec((1, _D), lambda i: (0, 0)),
        ],
        out_specs=out_spec,
        out_shape=out_shape,
    )


def kernel(x, edge_index, edge_attr, W0, b0, We0, W1, b1, We1, W2, b2, We2):
    src2 = edge_index[0].reshape(_NW, _NB2, _B2)
    dst3m = edge_index[1].reshape(_NW, _NCH2, _CH2, _B2)
    dst3 = edge_index[1].reshape(_NW, _NCH, _CH, _B)
    ea128 = jnp.pad(edge_attr, ((0, 0), (0, _D - _DE)))
    ea3 = ea128.reshape(_NW, _NB, _B, _D)

    (eap,) = _make_ea()(ea3, dst3)
    (acc0,) = _make_sc()(x, src2, dst3m)
    h0 = _make_tc(1.0, False)(acc0, eap, We0, W0, b0.reshape(1, _D))
    (acc1,) = _make_sc()(h0, src2, dst3m)
    h1 = _make_tc(2.0, False)(acc1, eap, We1, W1, b1.reshape(1, _D))
    (acc2,) = _make_sc()(h1, src2, dst3m)
    return _make_tc(2.0, True)(acc2, eap, We2, W2, b2.reshape(1, _D))


# speedup vs baseline: 6.0939x; 1.0629x over previous
import functools

import jax
import jax.numpy as jnp
from jax import lax
from jax.experimental import pallas as pl
from jax.experimental.pallas import tpu as pltpu
from jax.experimental.pallas import tpu_sc as plsc

_N = 10000
_E = 320000
_D = 128
_DE = 4
_NC, _NS = 2, 16
_NW = _NC * _NS
_EPW = _E // _NW
_B = 80
_NB = _EPW // _B
_CH = 25
_NCH = _NB // _CH
_B2 = 40
_NB2 = _EPW // _B2
_CH2 = 10
_NCH2 = _NB2 // _CH2
_NP = 10240
_RPT = _NP // _NS
_ZR = 8


def _sc_mesh():
    return plsc.VectorSubcoreMesh(core_axis_name="c", subcore_axis_name="s",
                                  num_cores=_NC, num_subcores=_NS)


_DP = 16


@functools.cache
def _make_sc():
    out_type = [jax.ShapeDtypeStruct((_NC, _NP, _D), jnp.float32)]
    scratch = [
        pltpu.VMEM_SHARED((_NP, _D), jnp.float32),
        pltpu.VMEM((_NB2, _B2), jnp.int32),
        pltpu.VMEM((2, _CH2, _B2), jnp.int32),
        pltpu.VMEM((_B2, _D), jnp.float32),
        pltpu.VMEM((_B2, _D), jnp.float32),
        pltpu.VMEM((_ZR, _D), jnp.float32),
        pltpu.SemaphoreType.DMA,
        pltpu.SemaphoreType.DMA,
        pltpu.SemaphoreType.DMA,
        pltpu.SemaphoreType.DMA,
    ]

    def body(g, src2, dst3, accout,
             acc_sh, src_v, dst_v, rows0, rows1, zbuf,
             sem0, sem1, ssem0, ssem1):
        c = lax.axis_index("c")
        s = lax.axis_index("s")
        wid = s * _NC + c
        rowbase = s * _RPT
        z = jnp.zeros((16,), jnp.float32)

        def drain(sem):
            pltpu.make_async_copy(g.at[src_v.at[0]], rows0, sem).wait()

        @pl.loop(0, _ZR)
        def _zero(i):
            for j in range(_D // 16):
                zbuf[i, pl.ds(j * 16, 16)] = z

        for k in range(_RPT // _ZR):
            pltpu.sync_copy(zbuf, acc_sh.at[pl.ds(rowbase + _ZR * k, _ZR)])

        pltpu.sync_copy(src2.at[wid], src_v)
        plsc.subcore_barrier()

        pltpu.async_copy(g.at[src_v.at[0]], rows0, sem0)

        @pl.loop(0, _NCH2)
        def _chunks(k):
            kp = k % 2
            pltpu.sync_copy(dst3.at[wid, k], dst_v.at[kp])

            @pl.loop(0, _CH2 // 2)
            def _pairs(m):
                jj = k * _CH2 + 2 * m

                @pl.when(jj >= 1)
                def _():
                    drain(ssem1)
                pltpu.async_copy(g.at[src_v.at[jj + 1]], rows1, sem1)
                drain(sem0)
                pltpu.async_copy(rows0, acc_sh.at[dst_v.at[kp, 2 * m]],
                                 ssem0, add=True)

                @pl.when(jj + 2 < _NB2)
                def _():
                    drain(ssem0)
                    pltpu.async_copy(g.at[src_v.at[jj + 2]], rows0, sem0)

                drain(sem1)
                pltpu.async_copy(rows1, acc_sh.at[dst_v.at[kp, 2 * m + 1]],
                                 ssem1, add=True)

        drain(ssem0)
        drain(ssem1)
        plsc.subcore_barrier()
        pltpu.sync_copy(acc_sh.at[pl.ds(rowbase, _RPT)],
                        accout.at[c, pl.ds(rowbase, _RPT)])

    return pl.kernel(body, out_type=out_type, mesh=_sc_mesh(),
                     scratch_types=scratch)


_RB = 1000


@functools.cache
def _make_tc(scale: float, pooled: bool):
    if pooled:
        out_shape = jax.ShapeDtypeStruct((1, _D), jnp.float32)
        out_spec = pl.BlockSpec((1, _D), lambda i: (0, 0))
    else:
        out_shape = jax.ShapeDtypeStruct((_N, _D), jnp.float32)
        out_spec = pl.BlockSpec((_RB, _D), lambda i: (i, 0))

    def body(acc_ref, ea_ref, We_ref, W_ref, b_ref, o_ref):
        a = acc_ref[0] + acc_ref[1]
        e = (ea_ref[0] + ea_ref[1])[:, :_DE]
        Wm = W_ref[...]
        WeW = jnp.dot(We_ref[...], Wm, preferred_element_type=jnp.float32)
        y = scale * jnp.dot(a, Wm, preferred_element_type=jnp.float32)
        y = y + jnp.dot(e, WeW, preferred_element_type=jnp.float32) + b_ref[...]
        h = jnp.where(y >= 0, y, 0.2 * y)
        if pooled:
            ps = jnp.sum(h, axis=0, keepdims=True)

            @pl.when(pl.program_id(0) == 0)
            def _first():
                o_ref[...] = ps

            @pl.when(pl.program_id(0) != 0)
            def _rest():
                o_ref[...] += ps
        else:
            o_ref[...] = h

    return pl.pallas_call(
        body,
        grid=(_N // _RB,),
        in_specs=[
            pl.BlockSpec((_NC, _RB, _D), lambda i: (0, i, 0)),
            pl.BlockSpec((_NC, _RB, _D), lambda i: (0, i, 0)),
            pl.BlockSpec((_DE, _D), lambda i: (0, 0)),
            pl.BlockSpec((_D, _D), lambda i: (0, 0)),
            pl.BlockSpec((1, _D), lambda i: (0, 0)),
        ],
        out_specs=out_spec,
        out_shape=out_shape,
    )


def kernel(x, edge_index, edge_attr, W0, b0, We0, W1, b1, We1, W2, b2, We2):
    src2 = edge_index[0].reshape(_NW, _NB2, _B2)
    dst3m = edge_index[1].reshape(_NW, _NCH2, _CH2, _B2)
    ea128 = jnp.pad(edge_attr, ((0, 0), (0, _D - _DE)))
    iota2 = jnp.arange(_E, dtype=jnp.int32).reshape(_NW, _NB2, _B2)

    (eap,) = _make_sc()(ea128, iota2, dst3m)
    (acc0,) = _make_sc()(x, src2, dst3m)
    h0 = _make_tc(1.0, False)(acc0, eap, We0, W0, b0.reshape(1, _D))
    (acc1,) = _make_sc()(h0, src2, dst3m)
    h1 = _make_tc(2.0, False)(acc1, eap, We1, W1, b1.reshape(1, _D))
    (acc2,) = _make_sc()(h1, src2, dst3m)
    return _make_tc(2.0, True)(acc2, eap, We2, W2, b2.reshape(1, _D))
